# baseline (speedup 1.0000x reference)
import functools

import jax
import jax.numpy as jnp
from jax import lax
from jax.experimental import pallas as pl
from jax.experimental.pallas import tpu as pltpu
from jax.experimental.pallas import tpu_sc as plsc

_TOKENS = 16384
_DIM = 4096
_E = 64
_K = 8
_BR = 1024
_NW = 32
_C = _TOKENS // _NW


def _matmul_kernel(x_ref, w_ref, out_ref):
    out_ref[...] = jax.lax.dot_general(
        x_ref[...], w_ref[...],
        dimension_numbers=(((1,), (1,)), ((), ())),
        preferred_element_type=jnp.float32,
    )


def _tc_logits(x, W):
    return pl.pallas_call(
        _matmul_kernel,
        grid=(_TOKENS // _BR,),
        in_specs=[
            pl.BlockSpec((_BR, _DIM), lambda i: (i, 0)),
            pl.BlockSpec((_E, _DIM), lambda i: (0, 0)),
        ],
        out_specs=pl.BlockSpec((_BR, _E), lambda i: (i, 0)),
        out_shape=jax.ShapeDtypeStruct((_TOKENS, _E), jnp.float32),
    )(x, W)


def _merge16(av, ai, bv, bi):
    rbv = lax.rev(bv, dimensions=(0,))
    rbi = lax.rev(bi, dimensions=(0,))
    take_a = (av > rbv) | ((av == rbv) & (ai < rbi))
    hv = jnp.where(take_a, av, rbv)
    hi = jnp.where(take_a, ai, rbi)
    return plsc.sort_key_val(hv, hi, descending=True)


def _sc_gate_kernel(lt_hbm, fw_hbm, idx_hbm, lt_v, fw_v, idx_v, sem):
    wid = lax.axis_index("s") * 2 + lax.axis_index("c")
    base = wid * _C
    pltpu.sync_copy(lt_hbm.at[pl.ds(base * _E, _C * _E)], lt_v)

    zeros16 = jnp.zeros((16,), jnp.float32)

    def _zero_body(i, carry):
        fw_v[pl.ds(i * 16, 16)] = zeros16
        return carry

    lax.fori_loop(0, _C * _E // 16, _zero_body, 0)

    lane = lax.iota(jnp.int32, 16)
    lane_lt8 = lane < 8

    def _tok_body(t, carry):
        svs = []
        sis = []
        for k in range(4):
            v = lt_v[pl.ds(t * _E + 16 * k, 16)]
            i = lane + (16 * k)
            sv, si = plsc.sort_key_val(v, i, descending=True)
            svs.append(sv)
            sis.append(si)
        m0v, m0i = _merge16(svs[0], sis[0], svs[1], sis[1])
        m1v, m1i = _merge16(svs[2], sis[2], svs[3], sis[3])
        tv, ti = _merge16(m0v, m0i, m1v, m1i)

        mval = jnp.max(tv)
        e = jnp.exp(tv - mval)
        em = jnp.where(lane_lt8, e, 0.0)
        s = jnp.sum(em)
        w = em / jnp.broadcast_to(s, (16,))

        flat_idx = t * _E + ti
        plsc.store_scatter(fw_v, [flat_idx], w, mask=lane_lt8)
        plsc.store_compressed(idx_v.at[pl.ds(t * _K, 16)], ti, mask=lane_lt8)
        return carry

    lax.fori_loop(0, _C, _tok_body, 0)

    pltpu.sync_copy(fw_v, fw_hbm.at[pl.ds(base * _E, _C * _E)])
    pltpu.sync_copy(idx_v.at[pl.ds(0, _C * _K)],
                    idx_hbm.at[pl.ds(base * _K, _C * _K)])


_sc_gate = functools.partial(
    pl.kernel,
    out_type=[
        jax.ShapeDtypeStruct((_TOKENS * _E,), jnp.float32),
        jax.ShapeDtypeStruct((_TOKENS * _K,), jnp.int32),
    ],
    mesh=plsc.VectorSubcoreMesh(core_axis_name="c", subcore_axis_name="s"),
    scratch_types=[
        pltpu.VMEM((_C * _E,), jnp.float32),
        pltpu.VMEM((_C * _E,), jnp.float32),
        pltpu.VMEM((_C * _K + 16,), jnp.int32),
        pltpu.SemaphoreType.DMA,
    ],
    compiler_params=pltpu.CompilerParams(needs_layout_passes=False),
)(_sc_gate_kernel)


@jax.jit
def kernel(x, W):
    logits = _tc_logits(x, W)
    fw_flat, idx_flat = _sc_gate(logits.reshape(_TOKENS * _E))
    return fw_flat.reshape(_TOKENS, _E), idx_flat.reshape(_TOKENS, _K)


# speedup vs baseline: 1.7872x; 1.7872x over previous
import jax
import jax.numpy as jnp
from jax.experimental import pallas as pl
from jax.experimental.pallas import tpu as pltpu

_TOKENS = 16384
_DIM = 4096
_E = 64
_K = 8
_BR = 1024


def _gate_kernel(x1_ref, x2_ref, w_ref, fw_ref, idx_ref):
    half = _DIM // 2
    logits_t = jax.lax.dot_general(
        w_ref[:, :half], x1_ref[...],
        dimension_numbers=(((1,), (1,)), ((), ())),
        preferred_element_type=jnp.float32,
    ) + jax.lax.dot_general(
        w_ref[:, half:], x2_ref[...],
        dimension_numbers=(((1,), (1,)), ((), ())),
        preferred_element_type=jnp.float32,
    )

    erow = jax.lax.broadcasted_iota(jnp.int32, logits_t.shape, 0)
    neg_inf = jnp.float32(-jnp.inf)

    cur = logits_t
    sel = jnp.zeros(logits_t.shape, jnp.bool_)
    idx_rows = []
    mx = None
    denom = None
    for t in range(_K):
        m = jnp.max(cur, axis=0, keepdims=True)
        is_max = cur == m
        idx = jnp.min(jnp.where(is_max, erow, _E), axis=0, keepdims=True)
        chosen = erow == idx
        sel = sel | chosen
        cur = jnp.where(chosen, neg_inf, cur)
        idx_rows.append(idx)
        if t == 0:
            mx = m
            denom = jnp.ones(m.shape, jnp.float32)
        else:
            denom = denom + jnp.exp(m - mx)

    inv = 1.0 / denom
    fw_t = jnp.where(sel, jnp.exp(logits_t - mx) * inv, 0.0)
    idx_t = jnp.concatenate(idx_rows, axis=0)
    fw_ref[...] = fw_t.T
    idx_ref[...] = idx_t.T


@jax.jit
def kernel(x, W):
    grid = (_TOKENS // _BR,)
    fw, idx = pl.pallas_call(
        _gate_kernel,
        grid=grid,
        in_specs=[
            pl.BlockSpec((_BR, _DIM // 2), lambda i: (i, 0)),
            pl.BlockSpec((_BR, _DIM // 2), lambda i: (i, 1)),
            pl.BlockSpec((_E, _DIM), lambda i: (0, 0)),
        ],
        out_specs=[
            pl.BlockSpec((_BR, _E), lambda i: (i, 0)),
            pl.BlockSpec((_BR, _K), lambda i: (i, 0)),
        ],
        out_shape=[
            jax.ShapeDtypeStruct((_TOKENS, _E), jnp.float32),
            jax.ShapeDtypeStruct((_TOKENS, _K), jnp.int32),
        ],
        compiler_params=pltpu.CompilerParams(
            vmem_limit_bytes=100 * 1024 * 1024,
        ),
    )(x, x, W)
    return fw, idx


# speedup vs baseline: 1.7885x; 1.0007x over previous
import jax
import jax.numpy as jnp
from jax.experimental import pallas as pl
from jax.experimental.pallas import tpu as pltpu

_TOKENS = 16384
_DIM = 4096
_E = 64
_K = 8
_BR = 1024


def _gate_kernel(x_ref, w_ref, fw_ref, idx_ref):
    logits_t = jax.lax.dot_general(
        w_ref[...], x_ref[...],
        dimension_numbers=(((1,), (1,)), ((), ())),
        preferred_element_type=jnp.float32,
    )

    erow = jax.lax.broadcasted_iota(jnp.int32, logits_t.shape, 0)
    neg_inf = jnp.float32(-jnp.inf)

    cur = logits_t
    sel = jnp.zeros(logits_t.shape, jnp.bool_)
    idx_rows = []
    mx = None
    denom = None
    for t in range(_K):
        m = jnp.max(cur, axis=0, keepdims=True)
        is_max = cur == m
        idx = jnp.min(jnp.where(is_max, erow, _E), axis=0, keepdims=True)
        chosen = erow == idx
        sel = sel | chosen
        cur = jnp.where(chosen, neg_inf, cur)
        idx_rows.append(idx)
        if t == 0:
            mx = m
            denom = jnp.ones(m.shape, jnp.float32)
        else:
            denom = denom + jnp.exp(m - mx)

    inv = 1.0 / denom
    fw_t = jnp.where(sel, jnp.exp(logits_t - mx) * inv, 0.0)
    idx_t = jnp.concatenate(idx_rows, axis=0)
    fw_ref[...] = fw_t.T
    idx_ref[...] = idx_t.T


@jax.jit
def kernel(x, W):
    grid = (_TOKENS // _BR,)
    fw, idx = pl.pallas_call(
        _gate_kernel,
        grid=grid,
        in_specs=[
            pl.BlockSpec((_BR, _DIM), lambda i: (i, 0)),
            pl.BlockSpec((_E, _DIM), lambda i: (0, 0)),
        ],
        out_specs=[
            pl.BlockSpec((_BR, _E), lambda i: (i, 0)),
            pl.BlockSpec((_BR, _K), lambda i: (i, 0)),
        ],
        out_shape=[
            jax.ShapeDtypeStruct((_TOKENS, _E), jnp.float32),
            jax.ShapeDtypeStruct((_TOKENS, _K), jnp.int32),
        ],
        compiler_params=pltpu.CompilerParams(
            vmem_limit_bytes=100 * 1024 * 1024,
        ),
    )(x, W)
    return fw, idx
